# baseline (speedup 1.0000x reference)
import functools

import jax
import jax.numpy as jnp
from jax import lax
from jax.experimental import pallas as pl
from jax.experimental.pallas import tpu as pltpu
from jax.experimental.pallas import tpu_sc as plsc

_V = 100000
_EMBED = 32
_NC = 2
_NS = 16
_NW = _NC * _NS
_CHUNK = 128
_TCPW = 24


def _make_flatten2():
    mesh = plsc.VectorSubcoreMesh(core_axis_name="c", subcore_axis_name="s")
    out_t = jax.ShapeDtypeStruct((_V // 4, _CHUNK), jnp.float32)

    @functools.partial(
        pl.kernel,
        mesh=mesh,
        compiler_params=pltpu.CompilerParams(use_tc_tiling_on_sc=True,
                                             needs_layout_passes=False),
        out_type=(out_t, out_t),
        scratch_types=[
            pltpu.VMEM((_EMBED, _CHUNK), jnp.float32),
            pltpu.VMEM((_EMBED, _CHUNK), jnp.float32),
        ],
    )
    def flatten2(ct_t, rt_t, ct_f, rt_f, buf, flat):
        wid = lax.axis_index("s") * _NC + lax.axis_index("c")
        iota = lax.iota(jnp.int32, 16)
        iota_hi = iota + 16

        def do_tile(tref, oref, tc, ncols):
            src_off = pl.multiple_of(tc * _CHUNK, _CHUNK)
            pltpu.sync_copy(tref.at[:, pl.ds(src_off, _CHUNK)], buf)

            @pl.loop(0, ncols, unroll=8)
            def _(c):
                cv = jnp.full((16,), c, jnp.int32)
                r = c // 4
                l = (c % 4) * _EMBED
                flat[r, pl.ds(l, 16)] = plsc.load_gather(buf, [iota, cv])
                flat[r, pl.ds(l + 16, 16)] = plsc.load_gather(buf, [iota_hi, cv])

            nrow = ncols * _EMBED // _CHUNK
            dst_off = pl.multiple_of(tc * _EMBED, _EMBED)
            pltpu.sync_copy(flat.at[pl.ds(0, nrow)],
                            oref.at[pl.ds(dst_off, nrow)])

        @pl.loop(0, _TCPW)
        def _(k):
            tc = k * _NW + wid
            do_tile(ct_t, ct_f, tc, _CHUNK)
            do_tile(rt_t, rt_f, tc, _CHUNK)

        @pl.when(wid < 13)
        def _():
            do_tile(ct_t, ct_f, 768 + wid, _CHUNK)
            do_tile(rt_t, rt_f, 768 + wid, _CHUNK)

        @pl.when(wid == 13)
        def _():
            do_tile(ct_t, ct_f, 781, _V - 781 * _CHUNK)
            do_tile(rt_t, rt_f, 781, _V - 781 * _CHUNK)

    return flatten2


def _make_gather2(B):
    bpw = B // _NW
    nchunk = bpw // _CHUNK
    mesh = plsc.VectorSubcoreMesh(core_axis_name="c", subcore_axis_name="s")

    @functools.partial(
        pl.kernel,
        mesh=mesh,
        compiler_params=pltpu.CompilerParams(use_tc_tiling_on_sc=False),
        out_type=jax.ShapeDtypeStruct((B, 2 * _EMBED), jnp.float32),
        scratch_types=[
            pltpu.VMEM((nchunk, _CHUNK), jnp.int32),
            pltpu.VMEM((2, _CHUNK, _EMBED), jnp.float32),
            pltpu.VMEM((2, _CHUNK, _EMBED), jnp.float32),
            pltpu.SemaphoreType.DMA,
            pltpu.SemaphoreType.DMA,
            pltpu.SemaphoreType.DMA,
        ],
    )
    def gather2(idx_hbm, ct_hbm, rt_hbm, out_hbm,
                idx_v, rows_c, rows_r, sem_c, sem_r, sem_w):
        wid = lax.axis_index("s") * _NC + lax.axis_index("c")
        base = wid * bpw
        pltpu.sync_copy(idx_hbm.at[wid], idx_v)
        gc = [None] * nchunk
        gr = [None] * nchunk
        wc = [None] * nchunk
        wr = [None] * nchunk

        def fire_writes(p):
            s = p % 2
            gc[p].wait()
            wc[p] = pltpu.async_copy(
                rows_c.at[s],
                out_hbm.at[pl.ds(base + p * _CHUNK, _CHUNK), pl.ds(0, _EMBED)],
                sem_w)
            gr[p].wait()
            wr[p] = pltpu.async_copy(
                rows_r.at[s],
                out_hbm.at[pl.ds(base + p * _CHUNK, _CHUNK),
                           pl.ds(_EMBED, _EMBED)],
                sem_w)

        for j in range(nchunk):
            s = j % 2
            if j >= 2:
                wc[j - 2].wait()
                wr[j - 2].wait()
            gc[j] = pltpu.async_copy(ct_hbm.at[idx_v.at[j]], rows_c.at[s], sem_c)
            gr[j] = pltpu.async_copy(rt_hbm.at[idx_v.at[j]], rows_r.at[s], sem_r)
            if j >= 1:
                fire_writes(j - 1)
        fire_writes(nchunk - 1)
        for p in (nchunk - 2, nchunk - 1):
            wc[p].wait()
            wr[p].wait()

    return gather2


_B = 1024 * 20
_FLATTEN2 = _make_flatten2()
_GATHER2 = _make_gather2(_B)


def _tc_copy_kernel(in_ref, out_ref):
    out_ref[...] = in_ref[...]


def _tc_copy_t(table_t):
    d, v = table_t.shape
    blk = 8
    return pl.pallas_call(
        _tc_copy_kernel,
        grid=(d // blk,),
        in_specs=[pl.BlockSpec((blk, v), lambda i: (i, 0))],
        out_specs=pl.BlockSpec((blk, v), lambda i: (i, 0)),
        out_shape=jax.ShapeDtypeStruct((d, v), table_t.dtype),
    )(table_t)


def kernel(program, concept_table, relation_table):
    batch, prog_len = program.shape[0], program.shape[1]
    idx = program[:, :, 1].astype(jnp.int32).reshape(_NW, -1, _CHUNK)
    ct_f, rt_f = _FLATTEN2(concept_table.T, relation_table.T)
    out = _GATHER2(idx, ct_f.reshape(_V, _EMBED), rt_f.reshape(_V, _EMBED))
    out = out.reshape(batch, prog_len, 2 * _EMBED)
    all_concepts = _tc_copy_t(concept_table.T).T
    return out, all_concepts


# speedup vs baseline: 1.8773x; 1.8773x over previous
import functools

import jax
import jax.numpy as jnp
from jax import lax
from jax.experimental import pallas as pl
from jax.experimental.pallas import tpu as pltpu
from jax.experimental.pallas import tpu_sc as plsc

_V = 100000
_EMBED = 32
_NC = 2
_NS = 16
_NW = _NC * _NS
_CHUNK = 128
_TCPW = 24


def _make_gather2(B):
    bpw = B // _NW
    nchunk = bpw // _CHUNK
    mesh = plsc.VectorSubcoreMesh(core_axis_name="c", subcore_axis_name="s")

    @functools.partial(
        pl.kernel,
        mesh=mesh,
        compiler_params=pltpu.CompilerParams(use_tc_tiling_on_sc=False),
        out_type=jax.ShapeDtypeStruct((B, 2 * _EMBED), jnp.float32),
        scratch_types=[
            pltpu.VMEM((nchunk, _CHUNK), jnp.int32),
            pltpu.VMEM((2, _CHUNK, _EMBED), jnp.float32),
            pltpu.VMEM((2, _CHUNK, _EMBED), jnp.float32),
            pltpu.SemaphoreType.DMA,
            pltpu.SemaphoreType.DMA,
            pltpu.SemaphoreType.DMA,
        ],
    )
    def gather2(idx_hbm, ct_hbm, rt_hbm, out_hbm,
                idx_v, rows_c, rows_r, sem_c, sem_r, sem_w):
        wid = lax.axis_index("s") * _NC + lax.axis_index("c")
        base = wid * bpw
        pltpu.sync_copy(idx_hbm.at[wid], idx_v)
        gc = [None] * nchunk
        gr = [None] * nchunk
        wc = [None] * nchunk
        wr = [None] * nchunk

        def fire_writes(p):
            s = p % 2
            gc[p].wait()
            wc[p] = pltpu.async_copy(
                rows_c.at[s],
                out_hbm.at[pl.ds(base + p * _CHUNK, _CHUNK), pl.ds(0, _EMBED)],
                sem_w)
            gr[p].wait()
            wr[p] = pltpu.async_copy(
                rows_r.at[s],
                out_hbm.at[pl.ds(base + p * _CHUNK, _CHUNK),
                           pl.ds(_EMBED, _EMBED)],
                sem_w)

        for j in range(nchunk):
            s = j % 2
            if j >= 2:
                wc[j - 2].wait()
                wr[j - 2].wait()
            gc[j] = pltpu.async_copy(ct_hbm.at[idx_v.at[j]], rows_c.at[s], sem_c)
            gr[j] = pltpu.async_copy(rt_hbm.at[idx_v.at[j]], rows_r.at[s], sem_r)
            if j >= 1:
                fire_writes(j - 1)
        fire_writes(nchunk - 1)
        for p in (nchunk - 2, nchunk - 1):
            wc[p].wait()
            wr[p].wait()

    return gather2


_B = 1024 * 20
_GATHER2 = _make_gather2(_B)


def _tc_copy_kernel(in_ref, out_ref):
    out_ref[...] = in_ref[...]


def _tc_copy_t(table_t):
    d, v = table_t.shape
    blk = 8
    return pl.pallas_call(
        _tc_copy_kernel,
        grid=(d // blk,),
        in_specs=[pl.BlockSpec((blk, v), lambda i: (i, 0))],
        out_specs=pl.BlockSpec((blk, v), lambda i: (i, 0)),
        out_shape=jax.ShapeDtypeStruct((d, v), table_t.dtype),
    )(table_t)


def kernel(program, concept_table, relation_table):
    batch, prog_len = program.shape[0], program.shape[1]
    idx = program[:, :, 1].astype(jnp.int32).reshape(_NW, -1, _CHUNK)
    out = _GATHER2(idx, concept_table, relation_table)
    out = out.reshape(batch, prog_len, 2 * _EMBED)
    all_concepts = _tc_copy_t(concept_table.T).T
    return out, all_concepts
